# baseline (speedup 1.0000x reference)
import jax
import jax.numpy as jnp
from jax import lax
from jax.experimental import pallas as pl
from jax.experimental.pallas import tpu as pltpu
from jax.experimental.pallas import tpu_sc as plsc

_NC = 2
_NS = 16
_NW = _NC * _NS
_IW = 128
_PK = 8


def _make_sc_pass(n_acc, d, e_rows, do_gather, r0=None):
    if r0 is None:
        r0 = e_rows // _NW
    r1 = e_rows // _NS - r0
    rpt = max(r0, r1)
    zr = n_acc // _NS
    pr = zr // _PK
    nb = 8
    mesh = plsc.VectorSubcoreMesh(core_axis_name="c", subcore_axis_name="s",
                                  num_cores=_NC, num_subcores=_NS)

    scratch = [
        pltpu.VMEM_SHARED((n_acc, d), jnp.float32),
        pltpu.VMEM((rpt, _IW), jnp.int32),
        pltpu.VMEM((zr, d), jnp.float32),
        pltpu.VMEM((pr, _PK * d), jnp.float32),
    ]
    if do_gather:
        scratch += [
            pltpu.VMEM_SHARED((n_acc, d), jnp.float32),
            pltpu.VMEM((rpt, _IW), jnp.int32),
            pltpu.VMEM((nb, _IW, d), jnp.float32),
        ] + [pltpu.SemaphoreType.DMA] * (2 * nb)
    else:
        scratch += [pltpu.VMEM((_IW, d), jnp.float32),
                    ] + [pltpu.SemaphoreType.DMA] * nb

    def body(*refs):
        if do_gather:
            (src2d, dst2d, featp, zrows, out0, out1,
             acc, didx, b16, b128, feat_sh, sidx, rows, *sems) = refs
        else:
            (dst2d, ones_hbm, zrows, out0, out1,
             acc, didx, b16, b128, rows, *sems) = refs
        c = lax.axis_index("c")
        s = lax.axis_index("s")
        rpt_dyn = jnp.where(c == 0, r0, r1)
        base = jnp.where(c == 0, s * r0, _NS * r0 + s * r1)

        def unpack_128_to_16(q, carry):
            for k in range(_PK):
                b16[q * _PK + k, :] = b128[q, pl.ds(k * d, d)]
            return carry

        def pack_16_to_128(q, carry):
            for k in range(_PK):
                b128[q, pl.ds(k * d, d)] = b16[q * _PK + k, :]
            return carry

        pltpu.sync_copy(zrows, acc.at[pl.ds(s * zr, zr)])
        pltpu.sync_copy(dst2d.at[pl.ds(base, rpt)], didx)
        if do_gather:
            pltpu.sync_copy(src2d.at[pl.ds(base, rpt)], sidx)
            pltpu.sync_copy(featp.at[pl.ds(s * pr, pr)], b128)
            lax.fori_loop(0, pr, unpack_128_to_16, 0)
            pltpu.sync_copy(b16, feat_sh.at[pl.ds(s * zr, zr)])
        else:
            pltpu.sync_copy(ones_hbm, rows)
        plsc.subcore_barrier()

        if do_gather:
            gsems, ssems = sems[:nb], sems[nb:]
            for b in range(nb):
                pltpu.async_copy(feat_sh.at[sidx.at[b]], rows.at[b], gsems[b])

            def group(g, carry):
                for b in range(nb):
                    j = g * nb + b
                    pltpu.make_async_copy(feat_sh.at[sidx.at[j]],
                                          rows.at[b], gsems[b]).wait()
                    pltpu.async_copy(rows.at[b], acc.at[didx.at[j]],
                                     ssems[b], add=True)
                for b in range(nb):
                    j = g * nb + b
                    pltpu.make_async_copy(rows.at[b], acc.at[didx.at[j]],
                                          ssems[b]).wait()
                    jn = j + nb

                    @pl.when(jn < rpt_dyn)
                    def _():
                        pltpu.async_copy(feat_sh.at[sidx.at[jn]],
                                         rows.at[b], gsems[b])
                return carry

            lax.fori_loop(0, rpt_dyn // nb, group, 0)
        else:
            for b in range(nb):
                pltpu.async_copy(rows, acc.at[didx.at[b]], sems[b], add=True)

            def group(g, carry):
                for b in range(nb):
                    j = g * nb + b
                    pltpu.make_async_copy(rows, acc.at[didx.at[j]],
                                          sems[b]).wait()
                    jn = j + nb

                    @pl.when(jn < rpt_dyn)
                    def _():
                        pltpu.async_copy(rows, acc.at[didx.at[jn]],
                                         sems[b], add=True)
                return carry

            lax.fori_loop(0, rpt_dyn // nb, group, 0)
        plsc.subcore_barrier()

        pltpu.sync_copy(acc.at[pl.ds(s * zr, zr)], b16)
        lax.fori_loop(0, pr, pack_16_to_128, 0)

        @pl.when(c == 0)
        def _():
            pltpu.sync_copy(b128, out0.at[pl.ds(s * pr, pr)])

        @pl.when(c == 1)
        def _():
            pltpu.sync_copy(b128, out1.at[pl.ds(s * pr, pr)])

    npk = n_acc // _PK
    return pl.kernel(
        body,
        out_type=(jax.ShapeDtypeStruct((npk, _PK * d), jnp.float32),
                  jax.ShapeDtypeStruct((npk, _PK * d), jnp.float32)),
        mesh=mesh,
        scratch_types=scratch,
        compiler_params=pltpu.CompilerParams(use_tc_tiling_on_sc=False),
    )


def _mm1_body(xg_ref, wb_ref, o_ref):
    o_ref[...] = jnp.dot(xg_ref[...], wb_ref[...],
                         preferred_element_type=jnp.float32)


def _norm_body(d0_ref, d1_ref, h_ref, dis_ref, hs_ref):
    deg = d0_ref[...] + d1_ref[...] + 1.0
    dis = lax.rsqrt(deg)
    dis_ref[...] = dis
    hs_ref[...] = dis * h_ref[...]


def _post1_body(a0_ref, a1_ref, dis_ref, hs_ref, b_ref, o_ref):
    dis = dis_ref[...]
    out1 = dis * (a0_ref[...] + a1_ref[...] + hs_ref[...]) + b_ref[...]
    o_ref[...] = dis * jnp.maximum(out1, 0.0)


def _zcomb_body(a0_ref, a1_ref, dis_ref, rs_ref, o_ref):
    o_ref[...] = dis_ref[...] * (a0_ref[...] + a1_ref[...] + rs_ref[...])


def _post2_body(z_ref, w_ref, b_ref, o_ref):
    logits = jnp.dot(z_ref[...], w_ref[...],
                     preferred_element_type=jnp.float32) + b_ref[...]
    m = jnp.max(logits, axis=1, keepdims=True)
    lse = jnp.log(jnp.sum(jnp.exp(logits - m), axis=1, keepdims=True)) + m
    o_ref[...] = logits - lse


def kernel(x, edge_index, W1, b1, W2, b2):
    n, d_in = x.shape
    e = edge_index.shape[1]
    d_hid = W1.shape[1]
    d_out = W2.shape[1]

    chunk = _IW * _NW * 8
    e_pad = -(-e // chunk) * chunk
    e_rows = e_pad // _IW
    n_acc = -(-n // 128) * 128 + 128
    npk = n_acc // _PK
    gp = 10
    bp = npk // gp
    dp = _PK * d_hid

    r0 = 96
    r1 = e_rows // _NS - r0
    arr_rows = e_rows + max(r0, r1)
    fill = arr_rows * _IW - e

    src = edge_index[0]
    dst = edge_index[1]
    dst2d = jnp.concatenate(
        [dst, jnp.full((fill,), n, dtype=jnp.int32)]).reshape(arr_rows, _IW)
    src2d = jnp.concatenate(
        [src, jnp.zeros((fill,), dtype=jnp.int32)]).reshape(arr_rows, _IW)

    zrows = jnp.zeros((n_acc // _NS, d_hid), dtype=jnp.float32)
    ones = jnp.ones((_IW, d_hid), dtype=jnp.float32)

    deg_pass = _make_sc_pass(n_acc, d_hid, e_rows, do_gather=False, r0=r0)
    agg_pass = _make_sc_pass(n_acc, d_hid, e_rows, do_gather=True, r0=r0)

    def packed_call(body, n_in, n_out):
        out_specs = [pl.BlockSpec((bp, dp), lambda i: (i, 0))] * n_out
        out_shape = [jax.ShapeDtypeStruct((npk, dp), jnp.float32)] * n_out
        if n_out == 1:
            out_specs, out_shape = out_specs[0], out_shape[0]
        return pl.pallas_call(
            body,
            grid=(gp,),
            in_specs=[pl.BlockSpec((bp, dp), lambda i: (i, 0))] * n_in,
            out_specs=out_specs,
            out_shape=out_shape,
        )

    deg0, deg1 = deg_pass(dst2d, ones, zrows)
    xg = jnp.pad(x, ((0, n_acc - n), (0, 0))).reshape(npk, _PK * d_in)
    w1big = jnp.kron(jnp.eye(_PK, dtype=jnp.float32), W1)
    hp = pl.pallas_call(
        _mm1_body,
        grid=(gp,),
        in_specs=[pl.BlockSpec((bp, _PK * d_in), lambda i: (i, 0)),
                  pl.BlockSpec((_PK * d_in, dp), lambda i: (0, 0))],
        out_specs=pl.BlockSpec((bp, dp), lambda i: (i, 0)),
        out_shape=jax.ShapeDtypeStruct((npk, dp), jnp.float32),
    )(xg, w1big)

    dis, hs = packed_call(_norm_body, 3, 2)(deg0, deg1, hp)

    a10, a11 = agg_pass(src2d, dst2d, hs, zrows)
    b1p = jnp.tile(b1, _PK).reshape(1, dp)
    rs = pl.pallas_call(
        _post1_body,
        grid=(gp,),
        in_specs=[pl.BlockSpec((bp, dp), lambda i: (i, 0))] * 4 +
                 [pl.BlockSpec((1, dp), lambda i: (0, 0))],
        out_specs=pl.BlockSpec((bp, dp), lambda i: (i, 0)),
        out_shape=jax.ShapeDtypeStruct((npk, dp), jnp.float32),
    )(a10, a11, dis, hs, b1p)

    a20, a21 = agg_pass(src2d, dst2d, rs, zrows)
    zp = packed_call(_zcomb_body, 4, 1)(a20, a21, dis, rs)
    z16 = zp.reshape(n_acc, d_hid)
    bn = 1000
    out = pl.pallas_call(
        _post2_body,
        grid=(n // bn,),
        in_specs=[pl.BlockSpec((bn, d_hid), lambda i: (i, 0)),
                  pl.BlockSpec((d_hid, d_out), lambda i: (0, 0)),
                  pl.BlockSpec((1, d_out), lambda i: (0, 0))],
        out_specs=pl.BlockSpec((bn, d_out), lambda i: (i, 0)),
        out_shape=jax.ShapeDtypeStruct((n, d_out), jnp.float32),
    )(z16, W2, b2.reshape(1, d_out))
    return out


# speedup vs baseline: 1.0238x; 1.0238x over previous
import jax
import jax.numpy as jnp
from jax import lax
from jax.experimental import pallas as pl
from jax.experimental.pallas import tpu as pltpu
from jax.experimental.pallas import tpu_sc as plsc

_NC = 2
_NS = 16
_NW = _NC * _NS
_IW = 128
_PK = 8


def _make_sc_pass(n_acc, d, e_rows, do_gather, r0=None):
    if r0 is None:
        r0 = e_rows // _NW
    r1 = e_rows // _NS - r0
    rpt = max(r0, r1)
    zr = n_acc // _NS
    pr = zr // _PK
    nb = 8
    mesh = plsc.VectorSubcoreMesh(core_axis_name="c", subcore_axis_name="s",
                                  num_cores=_NC, num_subcores=_NS)

    scratch = [
        pltpu.VMEM_SHARED((n_acc, d), jnp.float32),
        pltpu.VMEM((rpt, _IW), jnp.int32),
        pltpu.VMEM((zr, d), jnp.float32),
        pltpu.VMEM((pr, _PK * d), jnp.float32),
    ]
    if do_gather:
        scratch += [
            pltpu.VMEM_SHARED((n_acc, d), jnp.float32),
            pltpu.VMEM((rpt, _IW), jnp.int32),
            pltpu.VMEM((nb, _IW, d), jnp.float32),
        ] + [pltpu.SemaphoreType.DMA] * (2 * nb)
    else:
        scratch += [pltpu.VMEM((_IW, d), jnp.float32),
                    ] + [pltpu.SemaphoreType.DMA] * nb

    def body(*refs):
        if do_gather:
            (src2d, dst2d, featp, zrows, out0, out1,
             acc, didx, b16, b128, feat_sh, sidx, rows, *sems) = refs
        else:
            (dst2d, ones_hbm, zrows, out0, out1,
             acc, didx, b16, b128, rows, *sems) = refs
        c = lax.axis_index("c")
        s = lax.axis_index("s")
        rpt_dyn = jnp.where(c == 0, r0, r1)
        base = jnp.where(c == 0, s * r0, _NS * r0 + s * r1)

        def unpack_128_to_16(q, carry):
            for k in range(_PK):
                b16[q * _PK + k, :] = b128[q, pl.ds(k * d, d)]
            return carry

        def pack_16_to_128(q, carry):
            for k in range(_PK):
                b128[q, pl.ds(k * d, d)] = b16[q * _PK + k, :]
            return carry

        pltpu.sync_copy(zrows, acc.at[pl.ds(s * zr, zr)])
        pltpu.sync_copy(dst2d.at[pl.ds(base, rpt)], didx)
        if do_gather:
            pltpu.sync_copy(src2d.at[pl.ds(base, rpt)], sidx)
            pltpu.sync_copy(featp.at[pl.ds(s * pr, pr)], b128)
            lax.fori_loop(0, pr, unpack_128_to_16, 0)
            pltpu.sync_copy(b16, feat_sh.at[pl.ds(s * zr, zr)])
        else:
            pltpu.sync_copy(ones_hbm, rows)
        plsc.subcore_barrier()

        if do_gather:
            gsems, ssems = sems[:nb], sems[nb:]
            for b in range(nb):
                pltpu.async_copy(feat_sh.at[sidx.at[b]], rows.at[b], gsems[b])

            def group(g, carry):
                for b in range(nb):
                    j = g * nb + b
                    pltpu.make_async_copy(feat_sh.at[sidx.at[j]],
                                          rows.at[b], gsems[b]).wait()
                    pltpu.async_copy(rows.at[b], acc.at[didx.at[j]],
                                     ssems[b], add=True)
                for b in range(nb):
                    j = g * nb + b
                    pltpu.make_async_copy(rows.at[b], acc.at[didx.at[j]],
                                          ssems[b]).wait()
                    jn = j + nb

                    @pl.when(jn < rpt_dyn)
                    def _():
                        pltpu.async_copy(feat_sh.at[sidx.at[jn]],
                                         rows.at[b], gsems[b])
                return carry

            lax.fori_loop(0, rpt_dyn // nb, group, 0)
        else:
            for b in range(nb):
                pltpu.async_copy(rows, acc.at[didx.at[b]], sems[b], add=True)

            def group(g, carry):
                for b in range(nb):
                    j = g * nb + b
                    pltpu.make_async_copy(rows, acc.at[didx.at[j]],
                                          sems[b]).wait()
                    jn = j + nb

                    @pl.when(jn < rpt_dyn)
                    def _():
                        pltpu.async_copy(rows, acc.at[didx.at[jn]],
                                         sems[b], add=True)
                return carry

            lax.fori_loop(0, rpt_dyn // nb, group, 0)
        plsc.subcore_barrier()

        pltpu.sync_copy(acc.at[pl.ds(s * zr, zr)], b16)
        lax.fori_loop(0, pr, pack_16_to_128, 0)

        @pl.when(c == 0)
        def _():
            pltpu.sync_copy(b128, out0.at[pl.ds(s * pr, pr)])

        @pl.when(c == 1)
        def _():
            pltpu.sync_copy(b128, out1.at[pl.ds(s * pr, pr)])

    npk = n_acc // _PK
    return pl.kernel(
        body,
        out_type=(jax.ShapeDtypeStruct((npk, _PK * d), jnp.float32),
                  jax.ShapeDtypeStruct((npk, _PK * d), jnp.float32)),
        mesh=mesh,
        scratch_types=scratch,
        compiler_params=pltpu.CompilerParams(use_tc_tiling_on_sc=False),
    )


def _mm1_body(xg_ref, wb_ref, o_ref):
    o_ref[...] = jnp.dot(xg_ref[...], wb_ref[...],
                         preferred_element_type=jnp.float32)


def _norm_body(d0_ref, d1_ref, h_ref, dis_ref, hs_ref):
    deg = d0_ref[...] + d1_ref[...] + 1.0
    dis = lax.rsqrt(deg)
    dis_ref[...] = dis
    hs_ref[...] = dis * h_ref[...]


def _post1_body(a0_ref, a1_ref, dis_ref, hs_ref, b_ref, o_ref):
    dis = dis_ref[...]
    out1 = dis * (a0_ref[...] + a1_ref[...] + hs_ref[...]) + b_ref[...]
    o_ref[...] = dis * jnp.maximum(out1, 0.0)


def _zcomb_body(a0_ref, a1_ref, dis_ref, rs_ref, o_ref):
    o_ref[...] = dis_ref[...] * (a0_ref[...] + a1_ref[...] + rs_ref[...])


def _post2_body(z_ref, w_ref, b_ref, o_ref):
    logits = jnp.dot(z_ref[...], w_ref[...],
                     preferred_element_type=jnp.float32) + b_ref[...]
    m = jnp.max(logits, axis=1, keepdims=True)
    lse = jnp.log(jnp.sum(jnp.exp(logits - m), axis=1, keepdims=True)) + m
    o_ref[...] = logits - lse


def kernel(x, edge_index, W1, b1, W2, b2):
    n, d_in = x.shape
    e = edge_index.shape[1]
    d_hid = W1.shape[1]
    d_out = W2.shape[1]

    chunk = _IW * _NW * 8
    e_pad = -(-e // chunk) * chunk
    e_rows = e_pad // _IW
    n_acc = -(-n // 128) * 128 + 128
    npk = n_acc // _PK
    gp = 10
    bp = npk // gp
    dp = _PK * d_hid

    r0 = 88
    r1 = e_rows // _NS - r0
    arr_rows = e_rows + max(r0, r1)
    fill = arr_rows * _IW - e

    src = edge_index[0]
    dst = edge_index[1]
    dst2d = jnp.concatenate(
        [dst, jnp.full((fill,), n, dtype=jnp.int32)]).reshape(arr_rows, _IW)
    src2d = jnp.concatenate(
        [src, jnp.zeros((fill,), dtype=jnp.int32)]).reshape(arr_rows, _IW)

    zrows = jnp.zeros((n_acc // _NS, d_hid), dtype=jnp.float32)
    ones = jnp.ones((_IW, d_hid), dtype=jnp.float32)

    deg_pass = _make_sc_pass(n_acc, d_hid, e_rows, do_gather=False, r0=r0)
    agg_pass = _make_sc_pass(n_acc, d_hid, e_rows, do_gather=True, r0=r0)

    def packed_call(body, n_in, n_out):
        out_specs = [pl.BlockSpec((bp, dp), lambda i: (i, 0))] * n_out
        out_shape = [jax.ShapeDtypeStruct((npk, dp), jnp.float32)] * n_out
        if n_out == 1:
            out_specs, out_shape = out_specs[0], out_shape[0]
        return pl.pallas_call(
            body,
            grid=(gp,),
            in_specs=[pl.BlockSpec((bp, dp), lambda i: (i, 0))] * n_in,
            out_specs=out_specs,
            out_shape=out_shape,
        )

    deg0, deg1 = deg_pass(dst2d, ones, zrows)
    xg = jnp.pad(x, ((0, n_acc - n), (0, 0))).reshape(npk, _PK * d_in)
    w1big = jnp.kron(jnp.eye(_PK, dtype=jnp.float32), W1)
    hp = pl.pallas_call(
        _mm1_body,
        grid=(gp,),
        in_specs=[pl.BlockSpec((bp, _PK * d_in), lambda i: (i, 0)),
                  pl.BlockSpec((_PK * d_in, dp), lambda i: (0, 0))],
        out_specs=pl.BlockSpec((bp, dp), lambda i: (i, 0)),
        out_shape=jax.ShapeDtypeStruct((npk, dp), jnp.float32),
    )(xg, w1big)

    dis, hs = packed_call(_norm_body, 3, 2)(deg0, deg1, hp)

    a10, a11 = agg_pass(src2d, dst2d, hs, zrows)
    b1p = jnp.tile(b1, _PK).reshape(1, dp)
    rs = pl.pallas_call(
        _post1_body,
        grid=(gp,),
        in_specs=[pl.BlockSpec((bp, dp), lambda i: (i, 0))] * 4 +
                 [pl.BlockSpec((1, dp), lambda i: (0, 0))],
        out_specs=pl.BlockSpec((bp, dp), lambda i: (i, 0)),
        out_shape=jax.ShapeDtypeStruct((npk, dp), jnp.float32),
    )(a10, a11, dis, hs, b1p)

    a20, a21 = agg_pass(src2d, dst2d, rs, zrows)
    zp = packed_call(_zcomb_body, 4, 1)(a20, a21, dis, rs)
    z16 = zp.reshape(n_acc, d_hid)
    bn = 1000
    out = pl.pallas_call(
        _post2_body,
        grid=(n // bn,),
        in_specs=[pl.BlockSpec((bn, d_hid), lambda i: (i, 0)),
                  pl.BlockSpec((d_hid, d_out), lambda i: (0, 0)),
                  pl.BlockSpec((1, d_out), lambda i: (0, 0))],
        out_specs=pl.BlockSpec((bn, d_out), lambda i: (i, 0)),
        out_shape=jax.ShapeDtypeStruct((n, d_out), jnp.float32),
    )(z16, W2, b2.reshape(1, d_out))
    return out
